# baseline (speedup 1.0000x reference)
import functools

import jax
import jax.numpy as jnp
from jax import lax
from jax.experimental import pallas as pl
from jax.experimental.pallas import tpu as pltpu
from jax.experimental.pallas import tpu_sc as plsc

N_VOCAB = 49408
N_EMBD = 768
N_TOKENS = 77
BATCH = 1024

_LANES = 16
_NW = 32
_ROWS_PER_W = BATCH // _NW
_VECS_PER_ROW = N_EMBD // _LANES


def _sc_kernel(table_hbm, tokens_hbm, pos_hbm, out_hbm,
               idx_v, pos_v, buf_v, sem):
    wid = lax.axis_index("s") * 2 + lax.axis_index("c")
    row0 = wid * _ROWS_PER_W

    pltpu.sync_copy(tokens_hbm.at[pl.ds(row0, _ROWS_PER_W), :], idx_v)
    pltpu.sync_copy(pos_hbm, pos_v)

    def body(c, carry):
        b = row0 + c
        pltpu.async_copy(table_hbm.at[idx_v.at[c]], buf_v, sem).wait()

        def add_row(i, carry2):
            def add_vec(j, carry3):
                sl = pl.ds(j * _LANES, _LANES)
                buf_v[i, sl] = buf_v[i, sl] + pos_v[i, sl]
                return carry3
            return lax.fori_loop(0, _VECS_PER_ROW, add_vec, carry2,
                                 unroll=4)
        lax.fori_loop(0, N_TOKENS, add_row, 0)

        pltpu.sync_copy(buf_v, out_hbm.at[b])
        return carry

    lax.fori_loop(0, _ROWS_PER_W, body, 0)


def kernel(tokens, token_embedding, position_embedding):
    mesh = plsc.VectorSubcoreMesh(core_axis_name="c", subcore_axis_name="s")
    run = functools.partial(
        pl.kernel,
        mesh=mesh,
        out_type=jax.ShapeDtypeStruct((BATCH, N_TOKENS, N_EMBD),
                                      jnp.float32),
        scratch_types=[
            pltpu.VMEM((_ROWS_PER_W, N_TOKENS), jnp.int32),
            pltpu.VMEM((N_TOKENS, N_EMBD), jnp.float32),
            pltpu.VMEM((N_TOKENS, N_EMBD), jnp.float32),
            pltpu.SemaphoreType.DMA,
        ],
        compiler_params=pltpu.CompilerParams(use_tc_tiling_on_sc=False),
    )(_sc_kernel)
    return run(token_embedding, tokens, position_embedding)


# speedup vs baseline: 1.3574x; 1.3574x over previous
import functools

import jax
import jax.numpy as jnp
from jax import lax
from jax.experimental import pallas as pl
from jax.experimental.pallas import tpu as pltpu
from jax.experimental.pallas import tpu_sc as plsc

N_VOCAB = 49408
N_EMBD = 768
N_TOKENS = 77
BATCH = 1024

_LANES = 16
_NW = 32
_FLAT = BATCH * N_TOKENS
_PER_W = _FLAT // _NW
_K = 22
_NCH = _PER_W // _K
_NB = 4
_VECS = N_EMBD // _LANES


def _sc_kernel(table_hbm, tok2d_hbm, pos_hbm, out_hbm,
               idx_v, pos_v, b0, b1, b2, b3,
               sg0, sg1, sg2, sg3, sc0, sc1, sc2, sc3):
    bufs = (b0, b1, b2, b3)
    sgs = (sg0, sg1, sg2, sg3)
    scs = (sc0, sc1, sc2, sc3)

    wid = lax.axis_index("s") * 2 + lax.axis_index("c")
    ch0 = wid * _NCH

    pltpu.sync_copy(tok2d_hbm.at[pl.ds(ch0, _NCH), :], idx_v)
    pltpu.sync_copy(pos_hbm, pos_v)

    def g_desc(kl, buf, sem):
        return pltpu.make_async_copy(table_hbm.at[idx_v.at[kl]], buf, sem)

    def co_desc(kl, buf, sem):
        row = (ch0 + kl) * _K
        return pltpu.make_async_copy(buf, out_hbm.at[pl.ds(row, _K), :], sem)

    def add_pos(kl, buf):
        start_t = lax.rem((ch0 + kl) * _K, N_TOKENS)

        def row(i, carry):
            t = start_t + i
            t = jnp.where(t >= N_TOKENS, t - N_TOKENS, t)
            for j in range(_VECS):
                sl = pl.ds(j * _LANES, _LANES)
                plsc.addupdate(buf.at[i, sl], pos_v[t, sl])
            return carry

        lax.fori_loop(0, _K, row, 0)

    g_desc(0, bufs[0], sgs[0]).start()
    g_desc(1, bufs[1], sgs[1]).start()

    def group(g, carry):
        for beta in range(_NB):
            kl = g * _NB + beta
            nb = (beta + 2) % _NB

            @pl.when(kl + 2 < _NCH)
            def _():
                @pl.when(kl - 2 >= 0)
                def _():
                    co_desc(kl - 2, bufs[nb], scs[nb]).wait()
                g_desc(kl + 2, bufs[nb], sgs[nb]).start()

            g_desc(kl, bufs[beta], sgs[beta]).wait()
            add_pos(kl, bufs[beta])
            co_desc(kl, bufs[beta], scs[beta]).start()
        return carry

    lax.fori_loop(0, _NCH // _NB, group, 0)

    for beta in range(_NB):
        co_desc(_NCH - _NB + beta, bufs[beta], scs[beta]).wait()


def kernel(tokens, token_embedding, position_embedding):
    mesh = plsc.VectorSubcoreMesh(core_axis_name="c", subcore_axis_name="s")
    run = functools.partial(
        pl.kernel,
        mesh=mesh,
        out_type=jax.ShapeDtypeStruct((_FLAT, N_EMBD), jnp.float32),
        scratch_types=[
            pltpu.VMEM((_NCH, _K), jnp.int32),
            pltpu.VMEM((N_TOKENS, N_EMBD), jnp.float32),
        ] + [pltpu.VMEM((_K, N_EMBD), jnp.float32)] * _NB
          + [pltpu.SemaphoreType.DMA] * (2 * _NB),
        compiler_params=pltpu.CompilerParams(use_tc_tiling_on_sc=False),
    )(_sc_kernel)
    tok2d = tokens.reshape(_FLAT // _K, _K)
    out = run(token_embedding, tok2d, position_embedding)
    return out.reshape(BATCH, N_TOKENS, N_EMBD)


# speedup vs baseline: 1.8712x; 1.3785x over previous
import functools

import jax
import jax.numpy as jnp
from jax import lax
from jax.experimental import pallas as pl
from jax.experimental.pallas import tpu as pltpu
from jax.experimental.pallas import tpu_sc as plsc

N_VOCAB = 49408
N_EMBD = 768
N_TOKENS = 77
BATCH = 1024

_LANES = 16
_NW = 32
_ROWS_PER_W = BATCH // _NW
_C = 256
_NT = N_EMBD // _C
_CV = _C // _LANES
_MAIN = 72
_TAIL = 8
_NCH = _ROWS_PER_W * _NT


def _sc_kernel(table_hbm, tok_hbm, pos_hbm, out_hbm,
               idx_v, pos_v, b0, b1, b2, t0, t1, t2,
               sg0, sg1, sg2, sc0, sc1, sc2):
    bufs = (b0, b1, b2)
    tails = (t0, t1, t2)
    sgs = (sg0, sg1, sg2)
    scs = (sc0, sc1, sc2)

    wid = lax.axis_index("s") * 2 + lax.axis_index("c")
    row0 = wid * _ROWS_PER_W

    pltpu.sync_copy(tok_hbm.at[pl.ds(row0 * 80, _ROWS_PER_W * 80)], idx_v)
    pltpu.sync_copy(pos_hbm, pos_v)

    def g_main(c, beta):
        return pltpu.make_async_copy(
            table_hbm.at[idx_v.at[pl.ds(c * 80, _MAIN)],
                         pl.ds(beta * _C, _C)],
            bufs[beta].at[pl.ds(0, _MAIN), :], sgs[beta])

    def g_tail(c, beta):
        return pltpu.make_async_copy(
            table_hbm.at[idx_v.at[pl.ds(c * 80 + _MAIN, _TAIL)],
                         pl.ds(beta * _C, _C)],
            tails[beta], sgs[beta])

    def co(c, beta):
        return pltpu.make_async_copy(
            bufs[beta], out_hbm.at[row0 + c, :, pl.ds(beta * _C, _C)],
            scs[beta])

    def consume(c, beta):
        g_main(c, beta).wait()
        g_tail(c, beta).wait()
        buf, tail = bufs[beta], tails[beta]
        for i in range(N_TOKENS - _MAIN):
            for j in range(_CV):
                sl = pl.ds(j * _LANES, _LANES)
                buf[_MAIN + i, sl] = tail[i, sl]

        def addrow(i, carry):
            for j in range(_CV):
                plsc.addupdate(buf.at[i, pl.ds(j * _LANES, _LANES)],
                               pos_v[pl.ds(i * N_EMBD + beta * _C
                                           + j * _LANES, _LANES)])
            return carry
        lax.fori_loop(0, N_TOKENS, addrow, 0)
        co(c, beta).start()

    g_main(0, 0).start()
    g_tail(0, 0).start()
    g_main(0, 1).start()
    g_tail(0, 1).start()

    def body(c, carry):
        @pl.when(c > 0)
        def _():
            co(c - 1, 2).wait()
        g_main(c, 2).start()
        g_tail(c, 2).start()
        consume(c, 0)

        @pl.when(c < _ROWS_PER_W - 1)
        def _():
            co(c, 0).wait()
            g_main(c + 1, 0).start()
            g_tail(c + 1, 0).start()
        consume(c, 1)

        @pl.when(c < _ROWS_PER_W - 1)
        def _():
            co(c, 1).wait()
            g_main(c + 1, 1).start()
            g_tail(c + 1, 1).start()
        consume(c, 2)
        return carry

    lax.fori_loop(0, _ROWS_PER_W, body, 0)

    co(_ROWS_PER_W - 1, 0).wait()
    co(_ROWS_PER_W - 1, 1).wait()
    co(_ROWS_PER_W - 1, 2).wait()


def kernel(tokens, token_embedding, position_embedding):
    mesh = plsc.VectorSubcoreMesh(core_axis_name="c", subcore_axis_name="s")
    run = functools.partial(
        pl.kernel,
        mesh=mesh,
        out_type=jax.ShapeDtypeStruct((BATCH, N_TOKENS, N_EMBD),
                                      jnp.float32),
        scratch_types=[
            pltpu.VMEM((_ROWS_PER_W * 80,), jnp.int32),
            pltpu.VMEM((N_TOKENS * N_EMBD,), jnp.float32),
        ] + [pltpu.VMEM((N_TOKENS, _C), jnp.float32)] * 3
          + [pltpu.VMEM((_TAIL, _C), jnp.float32)] * 3
          + [pltpu.SemaphoreType.DMA] * 6,
    )(_sc_kernel)
    tok80 = jnp.concatenate(
        [tokens, jnp.tile(tokens[:, N_TOKENS - 1:], (1, 80 - N_TOKENS))],
        axis=1).reshape(-1)
    return run(token_embedding, tok80, position_embedding.reshape(-1))


# speedup vs baseline: 2.0563x; 1.0989x over previous
import functools

import jax
import jax.numpy as jnp
from jax import lax
from jax.experimental import pallas as pl
from jax.experimental.pallas import tpu as pltpu
from jax.experimental.pallas import tpu_sc as plsc

N_VOCAB = 49408
N_EMBD = 768
N_TOKENS = 77
BATCH = 1024

_LANES = 16
_NW = 32
_ROWS_PER_W = BATCH // _NW
_C = 256
_NT = N_EMBD // _C
_CV = _C // _LANES
_MAIN = 72
_TAIL = 8
_NB = 4
_POSC = N_TOKENS * _C


def _sc_kernel(table_hbm, tok_hbm, pos_hbm, out_hbm,
               idx_v, pos_v, pos_sh, b0, b1, b2, b3, t0, t1, t2, t3,
               sg0, sg1, sg2, sg3, sc0, sc1, sc2, sc3):
    bufs = (b0, b1, b2, b3)
    tails = (t0, t1, t2, t3)
    sgs = (sg0, sg1, sg2, sg3)
    scs = (sc0, sc1, sc2, sc3)

    sid = lax.axis_index("s")
    wid = sid * 2 + lax.axis_index("c")
    row0 = wid * _ROWS_PER_W

    pltpu.sync_copy(tok_hbm.at[pl.ds(row0 * 80, _ROWS_PER_W * 80)], idx_v)

    @pl.when(sid == 0)
    def _():
        pltpu.sync_copy(pos_hbm, pos_sh)
    plsc.subcore_barrier()

    def g_main(c, beta, tau):
        return pltpu.make_async_copy(
            table_hbm.at[idx_v.at[pl.ds(c * 80, _MAIN)],
                         pl.ds(tau * _C, _C)],
            bufs[beta].at[pl.ds(0, _MAIN), :], sgs[beta])

    def g_tail(c, beta, tau):
        return pltpu.make_async_copy(
            table_hbm.at[idx_v.at[pl.ds(c * 80 + _MAIN, _TAIL)],
                         pl.ds(tau * _C, _C)],
            tails[beta], sgs[beta])

    def co(c, beta, tau):
        return pltpu.make_async_copy(
            bufs[beta], out_hbm.at[row0 + c, :, pl.ds(tau * _C, _C)],
            scs[beta])

    def consume(c, beta, tau):
        g_main(c, beta, tau).wait()
        g_tail(c, beta, tau).wait()
        buf, tail = bufs[beta], tails[beta]
        for i in range(N_TOKENS - _MAIN):
            for j in range(_CV):
                sl = pl.ds(j * _LANES, _LANES)
                buf[_MAIN + i, sl] = tail[i, sl]

        def addrow(i, carry):
            for j in range(_CV):
                plsc.addupdate(buf.at[i, pl.ds(j * _LANES, _LANES)],
                               pos_v[pl.ds(i * _C + j * _LANES, _LANES)])
            return carry
        lax.fori_loop(0, N_TOKENS, addrow, 0, unroll=7)
        co(c, beta, tau).start()

    for tau in range(_NT):
        pltpu.sync_copy(pos_sh.at[pl.ds(tau * _POSC, _POSC)], pos_v)

        g_main(0, 0, tau).start()
        g_tail(0, 0, tau).start()
        g_main(1, 1, tau).start()
        g_tail(1, 1, tau).start()

        def grp(g, carry, tau=tau):
            for beta in range(_NB):
                m = g * _NB + beta
                nb = (beta + 2) % _NB
                if beta < 2:
                    @pl.when(g > 0)
                    def _():
                        co(m - 2, nb, tau).wait()
                    g_main(m + 2, nb, tau).start()
                    g_tail(m + 2, nb, tau).start()
                else:
                    @pl.when(g < _ROWS_PER_W // _NB - 1)
                    def _():
                        co(m - 2, nb, tau).wait()
                        g_main(m + 2, nb, tau).start()
                        g_tail(m + 2, nb, tau).start()
                consume(m, beta, tau)
            return carry

        lax.fori_loop(0, _ROWS_PER_W // _NB, grp, 0)

        for beta in range(_NB):
            co(_ROWS_PER_W - _NB + beta, beta, tau).wait()


def kernel(tokens, token_embedding, position_embedding):
    mesh = plsc.VectorSubcoreMesh(core_axis_name="c", subcore_axis_name="s")
    run = functools.partial(
        pl.kernel,
        mesh=mesh,
        out_type=jax.ShapeDtypeStruct((BATCH, N_TOKENS, N_EMBD),
                                      jnp.float32),
        scratch_types=[
            pltpu.VMEM((_ROWS_PER_W * 80,), jnp.int32),
            pltpu.VMEM((_POSC,), jnp.float32),
            pltpu.VMEM_SHARED((N_TOKENS * N_EMBD,), jnp.float32),
        ] + [pltpu.VMEM((N_TOKENS, _C), jnp.float32)] * _NB
          + [pltpu.VMEM((_TAIL, _C), jnp.float32)] * _NB
          + [pltpu.SemaphoreType.DMA] * (2 * _NB),
    )(_sc_kernel)
    tok80 = jnp.concatenate(
        [tokens, jnp.tile(tokens[:, N_TOKENS - 1:], (1, 80 - N_TOKENS))],
        axis=1).reshape(-1)
    pos3 = position_embedding.reshape(N_TOKENS, _NT, _C)
    pos3 = jnp.transpose(pos3, (1, 0, 2)).reshape(-1)
    return run(token_embedding, tok80, pos3)


# speedup vs baseline: 6.5743x; 3.1972x over previous
import functools

import jax
import jax.numpy as jnp
from jax import lax
from jax.experimental import pallas as pl
from jax.experimental.pallas import tpu as pltpu
from jax.experimental.pallas import tpu_sc as plsc

N_VOCAB = 49408
N_EMBD = 768
N_TOKENS = 77
BATCH = 1024

_LANES = 16
_NW = 32
_B = BATCH // _NW
_VECS = N_EMBD // _LANES
_NB = 4
_NGRP = 19


def _sc_kernel(table_hbm, tok_hbm, pos_hbm, out_hbm,
               idxT_v, pos_sh, b0, b1, b2, b3, p0, p1, p2, p3,
               sg0, sg1, sg2, sg3, sc0, sc1, sc2, sc3,
               sp0, sp1, sp2, sp3):
    bufs = (b0, b1, b2, b3)
    prs = (p0, p1, p2, p3)
    sgs = (sg0, sg1, sg2, sg3)
    scs = (sc0, sc1, sc2, sc3)
    sps = (sp0, sp1, sp2, sp3)

    sid = lax.axis_index("s")
    wid = sid * 2 + lax.axis_index("c")
    bat0 = wid * _B

    pltpu.sync_copy(tok_hbm.at[pl.ds(wid * _B * N_TOKENS, _B * N_TOKENS)],
                    idxT_v)

    @pl.when(sid == 0)
    def _():
        pltpu.sync_copy(pos_hbm, pos_sh)
    plsc.subcore_barrier()

    def g_desc(t, beta):
        return pltpu.make_async_copy(
            table_hbm.at[idxT_v.at[pl.ds(t * _B, _B)]], bufs[beta],
            sgs[beta])

    def p_desc(t, beta):
        return pltpu.make_async_copy(
            pos_sh.at[pl.ds(t * N_EMBD, N_EMBD)], prs[beta], sps[beta])

    def co_desc(t, beta):
        return pltpu.make_async_copy(
            bufs[beta], out_hbm.at[t, pl.ds(bat0, _B), :], scs[beta])

    def issue(t, beta):
        g_desc(t, beta).start()
        p_desc(t, beta).start()

    def consume(t, beta):
        g_desc(t, beta).wait()
        p_desc(t, beta).wait()
        buf, pr = bufs[beta], prs[beta]

        def addvec(j, carry):
            pvec = pr[pl.ds(j * _LANES, _LANES)]
            for i in range(_B):
                plsc.addupdate(buf.at[i, pl.ds(j * _LANES, _LANES)], pvec)
            return carry
        lax.fori_loop(0, _VECS, addvec, 0, unroll=2)
        co_desc(t, beta).start()

    issue(0, 0)
    issue(1, 1)

    def grp(g, carry):
        for beta in range(_NB):
            m = g * _NB + beta
            nb = (beta + 2) % _NB
            if beta < 2:
                @pl.when(g > 0)
                def _():
                    co_desc(m - 2, nb).wait()
                issue(m + 2, nb)
            elif beta == 2:
                co_desc(m - 2, nb).wait()
                issue(m + 2, nb)
            else:
                co_desc(m - 2, nb).wait()
                @pl.when(g < _NGRP - 1)
                def _():
                    issue(m + 2, nb)
            consume(m, beta)
        return carry

    lax.fori_loop(0, _NGRP, grp, 0)

    co_desc(74, 2).wait()
    consume(76, 0)
    co_desc(75, 3).wait()
    co_desc(76, 0).wait()


def kernel(tokens, token_embedding, position_embedding):
    mesh = plsc.VectorSubcoreMesh(core_axis_name="c", subcore_axis_name="s")
    run = functools.partial(
        pl.kernel,
        mesh=mesh,
        out_type=jax.ShapeDtypeStruct((N_TOKENS, BATCH, N_EMBD),
                                      jnp.float32),
        scratch_types=[
            pltpu.VMEM((_B * N_TOKENS,), jnp.int32),
            pltpu.VMEM_SHARED((N_TOKENS * N_EMBD,), jnp.float32),
        ] + [pltpu.VMEM((_B, N_EMBD), jnp.float32)] * _NB
          + [pltpu.VMEM((N_EMBD,), jnp.float32)] * _NB
          + [pltpu.SemaphoreType.DMA] * (3 * _NB),
    )(_sc_kernel)
    tokT = jnp.transpose(tokens.reshape(_NW, _B, N_TOKENS),
                         (0, 2, 1)).reshape(-1)
    out3 = run(token_embedding, tokT, position_embedding.reshape(-1))
    return jnp.transpose(out3, (1, 0, 2))
